# Initial kernel scaffold; baseline (speedup 1.0000x reference)
import jax
import jax.numpy as jnp
from jax.experimental import pallas as pl


def kernel(x, adj, proj_W, proj_b, W1, b1, W2, b2, pred_W, pred_b):
    raise NotImplementedError("write your pallas kernel here")


# speedup vs baseline: 185.0633x; 185.0633x over previous
import functools

import jax
import jax.numpy as jnp
from jax import lax
from jax.experimental import pallas as pl
from jax.experimental.pallas import tpu as pltpu

N = 1024
D_IN = 256
H = 64
IC = 8


def _pna_body(x_ref, adj_ref, pw_ref, pb_ref, w1_ref, b1_ref, w2_ref, b2_ref,
              predw_ref, predb_ref, out_ref, h_ref):
    x = x_ref[0]
    adj = adj_ref[0]

    h = jnp.dot(x, pw_ref[...], preferred_element_type=jnp.float32) + pb_ref[...]
    h_ref[...] = h

    row = lax.broadcasted_iota(jnp.int32, (N, N), 0)
    col = lax.broadcasted_iota(jnp.int32, (N, N), 1)
    mask = (adj != 0) | (row == col)
    af = mask.astype(jnp.float32)

    deg = jnp.sum(af, axis=0)
    degc = jnp.maximum(deg, 1.0)[:, None]

    cdims = (((0,), (0,)), ((), ()))
    ssum = lax.dot_general(af, h, cdims, preferred_element_type=jnp.float32)
    ssq = lax.dot_general(af, h * h, cdims, preferred_element_type=jnp.float32)

    neg = jnp.full((H, N), -jnp.inf, dtype=jnp.float32)
    pos = jnp.full((H, N), jnp.inf, dtype=jnp.float32)
    cj = lax.broadcasted_iota(jnp.int32, (1, N), 1)

    def step(c, carry):
        mx, mn = carry
        hblk = h_ref[pl.ds(c * IC, IC), :]
        hT = jnp.transpose(hblk)
        for k in range(IC):
            i = c * IC + k
            arow = adj_ref[0, pl.ds(i, 1), :]
            m = (arow != 0.0) | (cj == i)
            hcol = hT[:, k:k + 1]
            mx = jnp.maximum(mx, jnp.where(m, hcol, -jnp.inf))
            mn = jnp.minimum(mn, jnp.where(m, hcol, jnp.inf))
        return mx, mn

    smaxT, sminT = lax.fori_loop(0, N // IC, step, (neg, pos))
    smax = jnp.transpose(smaxT)
    smin = jnp.transpose(sminT)

    mean = ssum / degc
    var = jnp.maximum(ssq / degc - mean * mean, 0.0)
    std = jnp.sqrt(var + 1e-5)
    aggs = jnp.concatenate([mean, smax, smin, std], axis=1)

    logd = jnp.log(deg + 1.0)
    delta = jnp.mean(logd)
    amp = (logd / delta)[:, None]
    att = (delta / jnp.maximum(logd, 1e-5))[:, None]
    scaled = jnp.concatenate([aggs, aggs * amp, aggs * att], axis=1)
    z = jnp.concatenate([h, scaled], axis=1)

    z = jnp.maximum(jnp.dot(z, w1_ref[...], preferred_element_type=jnp.float32)
                    + b1_ref[...], 0.0)
    z = jnp.dot(z, w2_ref[...], preferred_element_type=jnp.float32) + b2_ref[...]
    out = jnp.dot(z, predw_ref[...], preferred_element_type=jnp.float32) + predb_ref[...]
    out_ref[0] = out


@jax.jit
def kernel(x, adj, proj_W, proj_b, W1, b1, W2, b2, pred_W, pred_b):
    bs = x.shape[0]
    full = lambda s: pl.BlockSpec(s, lambda i: (0,) * len(s))
    return pl.pallas_call(
        _pna_body,
        grid=(bs,),
        in_specs=[
            pl.BlockSpec((1, N, D_IN), lambda i: (i, 0, 0)),
            pl.BlockSpec((1, N, N), lambda i: (i, 0, 0)),
            full((D_IN, H)),
            full((H,)),
            full((13 * H, 2 * H)),
            full((2 * H,)),
            full((2 * H, H)),
            full((H,)),
            full((H, 1)),
            full((1,)),
        ],
        out_specs=pl.BlockSpec((1, N, 1), lambda i: (i, 0, 0)),
        out_shape=jax.ShapeDtypeStruct((bs, N, 1), jnp.float32),
        scratch_shapes=[pltpu.VMEM((N, H), jnp.float32)],
    )(x, adj, proj_W, proj_b, W1, b1, W2, b2, pred_W, pred_b)


# speedup vs baseline: 411.2579x; 2.2223x over previous
import functools

import jax
import jax.numpy as jnp
from jax import lax
from jax.experimental import pallas as pl
from jax.experimental.pallas import tpu as pltpu

N = 1024
D_IN = 256
H = 64
IC = 16


def _pna_body(x_ref, adj_ref, pw_ref, pb_ref, w1_ref, b1_ref, w2_ref, b2_ref,
              predw_ref, predb_ref, out_ref, h_ref, aug_ref):
    x = x_ref[0]
    adj = adj_ref[0]

    h = jnp.dot(x, pw_ref[...], preferred_element_type=jnp.float32) + pb_ref[...]
    h_ref[...] = h

    row = lax.broadcasted_iota(jnp.int32, (N, N), 0)
    col = lax.broadcasted_iota(jnp.int32, (N, N), 1)
    mask = (adj != 0) | (row == col)
    af = mask.astype(jnp.float32)
    aug_ref[...] = jnp.where(mask, 0.0, -1e30).astype(jnp.bfloat16)

    deg = jnp.sum(af, axis=0)
    degc = jnp.maximum(deg, 1.0)[:, None]

    cdims = (((0,), (0,)), ((), ()))
    ssum = lax.dot_general(af, h, cdims, preferred_element_type=jnp.float32)
    ssq = lax.dot_general(af, h * h, cdims, preferred_element_type=jnp.float32)

    bneg = jnp.full((H, N), -jnp.inf, dtype=jnp.bfloat16)
    bpos = jnp.full((H, N), jnp.inf, dtype=jnp.bfloat16)

    def step(c, carry):
        mx, mn = carry
        base = pl.multiple_of(c * IC, IC)
        augblk = aug_ref[pl.ds(base, IC), :]
        hblk = h_ref[pl.ds(base, IC), :].astype(jnp.bfloat16)
        hT = jnp.transpose(hblk)
        for k in range(IC):
            pen = augblk[k:k + 1, :]
            hcol = hT[:, k:k + 1]
            mx = jnp.maximum(mx, hcol + pen)
            mn = jnp.minimum(mn, hcol - pen)
        return mx, mn

    smaxT, sminT = lax.fori_loop(0, N // IC, step, (bneg, bpos))
    smax = jnp.transpose(smaxT).astype(jnp.float32)
    smin = jnp.transpose(sminT).astype(jnp.float32)

    mean = ssum / degc
    var = jnp.maximum(ssq / degc - mean * mean, 0.0)
    std = jnp.sqrt(var + 1e-5)
    aggs = jnp.concatenate([mean, smax, smin, std], axis=1)

    logd = jnp.log(deg + 1.0)
    delta = jnp.mean(logd)
    amp = (logd / delta)[:, None]
    att = (delta / jnp.maximum(logd, 1e-5))[:, None]
    scaled = jnp.concatenate([aggs, aggs * amp, aggs * att], axis=1)
    z = jnp.concatenate([h, scaled], axis=1)

    z = jnp.maximum(jnp.dot(z, w1_ref[...], preferred_element_type=jnp.float32)
                    + b1_ref[...], 0.0)
    z = jnp.dot(z, w2_ref[...], preferred_element_type=jnp.float32) + b2_ref[...]
    out = jnp.dot(z, predw_ref[...], preferred_element_type=jnp.float32) + predb_ref[...]
    out_ref[0] = out


@jax.jit
def kernel(x, adj, proj_W, proj_b, W1, b1, W2, b2, pred_W, pred_b):
    bs = x.shape[0]
    full = lambda s: pl.BlockSpec(s, lambda i: (0,) * len(s))
    return pl.pallas_call(
        _pna_body,
        grid=(bs,),
        in_specs=[
            pl.BlockSpec((1, N, D_IN), lambda i: (i, 0, 0)),
            pl.BlockSpec((1, N, N), lambda i: (i, 0, 0)),
            full((D_IN, H)),
            full((H,)),
            full((13 * H, 2 * H)),
            full((2 * H,)),
            full((2 * H, H)),
            full((H,)),
            full((H, 1)),
            full((1,)),
        ],
        out_specs=pl.BlockSpec((1, N, 1), lambda i: (i, 0, 0)),
        out_shape=jax.ShapeDtypeStruct((bs, N, 1), jnp.float32),
        scratch_shapes=[pltpu.VMEM((N, H), jnp.float32),
                        pltpu.VMEM((N, N), jnp.bfloat16)],
    )(x, adj, proj_W, proj_b, W1, b1, W2, b2, pred_W, pred_b)
